# baseline (speedup 1.0000x reference)
import functools

import jax
import jax.numpy as jnp
from jax import lax
from jax.experimental import pallas as pl
from jax.experimental.pallas import tpu as pltpu
from jax.experimental.pallas import tpu_sc as plsc

D_MODEL = 1024
SEQ = 8192
NUM_CORES = 2
NUM_SUBCORES = 16
NUM_WORKERS = NUM_CORES * NUM_SUBCORES

K_CHUNKS = 4
CHUNK_SEQ = SEQ // K_CHUNKS
ROWS_PER_WORKER = CHUNK_SEQ // NUM_WORKERS
S_BLK = 512
BLKS_PER_CHUNK = CHUNK_SEQ // S_BLK

_SC_MESH = plsc.VectorSubcoreMesh(core_axis_name="c", subcore_axis_name="s")


@functools.partial(
    pl.kernel,
    mesh=_SC_MESH,
    out_type=jax.ShapeDtypeStruct((CHUNK_SEQ, D_MODEL), jnp.float32),
    scratch_types=[
        pltpu.VMEM((ROWS_PER_WORKER,), jnp.int32),
        pltpu.VMEM((ROWS_PER_WORKER, D_MODEL), jnp.float32),
        pltpu.SemaphoreType.DMA,
    ],
)
def _gather_chunk_sc(table_hbm, idx_hbm, out_hbm, idx_v, rows_v, sem):
    wid = lax.axis_index("s") * NUM_CORES + lax.axis_index("c")
    base = wid * ROWS_PER_WORKER
    pltpu.sync_copy(idx_hbm.at[pl.ds(base, ROWS_PER_WORKER)], idx_v)
    pltpu.async_copy(table_hbm.at[idx_v], rows_v, sem).wait()
    pltpu.sync_copy(rows_v, out_hbm.at[pl.ds(base, ROWS_PER_WORKER)])


def _add_body(x_ref, p_ref, o_ref):
    o_ref[...] = x_ref[...] + p_ref[...][None, :, :]


def _add_body_carry(c_ref, x_ref, p_ref, o_ref):
    del c_ref
    o_ref[...] = x_ref[...] + p_ref[...][None, :, :]


def _add_chunk_tc(chunk_idx, carry, x, pos):
    b = x.shape[0]
    base_blk = chunk_idx * BLKS_PER_CHUNK

    def xmap(j, base_blk=base_blk):
        return (0, base_blk + j, 0)

    x_spec = pl.BlockSpec((b, S_BLK, D_MODEL), xmap)
    p_spec = pl.BlockSpec((S_BLK, D_MODEL), lambda j: (j, 0))
    o_spec = pl.BlockSpec((b, S_BLK, D_MODEL), xmap)
    out_shape = jax.ShapeDtypeStruct(x.shape, x.dtype)
    if carry is None:
        return pl.pallas_call(
            _add_body,
            grid=(BLKS_PER_CHUNK,),
            in_specs=[x_spec, p_spec],
            out_specs=o_spec,
            out_shape=out_shape,
        )(x, pos)
    return pl.pallas_call(
        _add_body_carry,
        grid=(BLKS_PER_CHUNK,),
        in_specs=[pl.BlockSpec(memory_space=pl.ANY), x_spec, p_spec],
        out_specs=o_spec,
        out_shape=out_shape,
        input_output_aliases={0: 0},
    )(carry, x, pos)


def kernel(x, region_ids, pos_table):
    ids = region_ids.astype(jnp.int32)
    pos_chunks = [
        _gather_chunk_sc(pos_table, ids[i * CHUNK_SEQ:(i + 1) * CHUNK_SEQ])
        for i in range(K_CHUNKS)
    ]
    carry = None
    for i in range(K_CHUNKS):
        carry = _add_chunk_tc(i, carry, x, pos_chunks[i])
    return carry
